# baseline (speedup 1.0000x reference)
import jax
import jax.numpy as jnp
from jax.experimental import pallas as pl
from jax.experimental.pallas import tpu as pltpu

HPW = 128
NBUF = 8
GRP = 8


def _round_up(v, m):
    return ((v + m - 1) // m) * m


def _pad_2d(a, rows, cols):
    out = jnp.zeros((rows, cols), jnp.float32)
    return out.at[: a.shape[0], : a.shape[1]].set(a.astype(jnp.float32))


def _make_body(n_steps, tiles_per_step, tile_e, n_nodes, n_graphs):
    f32 = jnp.float32

    def body(
        src_ref, dst_ref,
        x_ref,
        ea_ref,
        batch_ref,
        wmx_ref, wme_ref, bm_ref,
        w1_ref, b1_ref, w2_ref, b2_ref, w3_ref, b3_ref,
        w4_ref, b4_ref, w5_ref, b5_ref,
        out_ref,
        xw3_ref,
        eaw_ref,
        *hbufs,
    ):
        t = pl.program_id(0)

        @pl.when(t == 0)
        def _init():
            xw = jnp.dot(x_ref[...], wmx_ref[...], preferred_element_type=f32)
            xw3_ref[...] = xw.reshape(xw3_ref.shape)
            for hb in hbufs:
                hb[...] = jnp.zeros_like(hb)

        eaw_ref[...] = (
            jnp.dot(ea_ref[...], wme_ref[...], preferred_element_type=f32)
            + bm_ref[...]
        )

        def tile_body(k, carry):
            koff = pl.multiple_of(k * tile_e, tile_e)
            base = (t * tiles_per_step + k) * tile_e

            for g in range(tile_e // GRP):
                chunk = eaw_ref[pl.ds(koff + g * GRP, GRP), :]
                msgs = []
                dsts = []
                for j in range(GRP):
                    e = base + g * GRP + j
                    s = src_ref[e]
                    dsts.append(dst_ref[e])
                    xrow = xw3_ref[s, 0]
                    msgs.append(jnp.maximum(xrow + chunk[j], 0.0))
                loads = [hbufs[j][dsts[j], 0] for j in range(GRP)]
                for j in range(GRP):
                    hbufs[j][dsts[j], 0] = loads[j] + msgs[j]
            return carry

        jax.lax.fori_loop(0, tiles_per_step, tile_body, 0)

        @pl.when(t == n_steps - 1)
        def _finalize():
            acc = hbufs[0][...]
            for hb in hbufs[1:]:
                acc = acc + hb[...]
            h = acc.reshape(acc.shape[0], HPW)[:n_nodes]
            h = jnp.maximum(
                jnp.dot(h, w1_ref[...], preferred_element_type=f32) + b1_ref[...], 0.0)
            h = jnp.maximum(
                jnp.dot(h, w2_ref[...], preferred_element_type=f32) + b2_ref[...], 0.0)
            h = jnp.maximum(
                jnp.dot(h, w3_ref[...], preferred_element_type=f32) + b3_ref[...], 0.0)

            gid = jax.lax.broadcasted_iota(jnp.int32, (n_graphs, n_nodes), 0)
            pool = (gid == batch_ref[...]).astype(f32)
            gvec = jnp.dot(pool, h, preferred_element_type=f32)

            gvec = jnp.maximum(
                jnp.dot(gvec, w4_ref[...], preferred_element_type=f32) + b4_ref[...], 0.0)
            out_ref[...] = (
                jnp.dot(gvec, w5_ref[...], preferred_element_type=f32) + b5_ref[...])

    return body


def _forward(node_attr, edge_attr, edge_index, batch, params, num_graphs,
             tile_e=512, tiles_per_step=16):
    f32 = jnp.float32
    n_nodes, din = node_attr.shape
    n_edges, de = edge_attr.shape
    g = int(num_graphs)

    te = max(GRP, _round_up(min(tile_e, n_edges), GRP))
    tps = tiles_per_step
    e_pad = _round_up(n_edges, te * tps)
    n_steps = e_pad // (te * tps)

    x = node_attr.astype(f32)
    ea = jnp.zeros((e_pad, de), f32).at[:n_edges].set(edge_attr.astype(f32))
    src = jnp.zeros((e_pad,), jnp.int32).at[:n_edges].set(
        edge_index[0].astype(jnp.int32))
    dst = jnp.full((e_pad,), n_nodes, jnp.int32).at[:n_edges].set(
        edge_index[1].astype(jnp.int32))
    batch2 = batch.astype(jnp.int32).reshape(1, n_nodes)

    wmx = _pad_2d(params["wmx"], din, HPW)
    wme = _pad_2d(params["wme"], de, HPW)
    bm = _pad_2d(params["bm"], 1, HPW)
    w1 = _pad_2d(params["w1"], HPW, HPW); b1 = _pad_2d(params["b1"], 1, HPW)
    w2 = _pad_2d(params["w2"], HPW, HPW); b2 = _pad_2d(params["b2"], 1, HPW)
    w3 = _pad_2d(params["w3"], HPW, HPW); b3 = _pad_2d(params["b3"], 1, HPW)
    w4 = _pad_2d(params["w4"], HPW, HPW); b4 = _pad_2d(params["b4"], 1, HPW)
    w5 = _pad_2d(params["w5"], HPW, HPW); b5 = _pad_2d(params["b5"], 1, HPW)

    def cspec(shape):
        return pl.BlockSpec(shape, lambda t, *_: tuple(0 for _ in shape))

    in_specs = [
        cspec((n_nodes, din)),
        pl.BlockSpec((te * tps, de), lambda t, *_: (t, 0)),
        cspec((1, n_nodes)),
        cspec((din, HPW)), cspec((de, HPW)), cspec((1, HPW)),
        cspec((HPW, HPW)), cspec((1, HPW)),
        cspec((HPW, HPW)), cspec((1, HPW)),
        cspec((HPW, HPW)), cspec((1, HPW)),
        cspec((HPW, HPW)), cspec((1, HPW)),
        cspec((HPW, HPW)), cspec((1, HPW)),
    ]

    scratch = [
        pltpu.VMEM((n_nodes, 1, HPW), f32),
        pltpu.VMEM((te * tps, HPW), f32),
    ] + [pltpu.VMEM((n_nodes + 8, 1, HPW), f32) for _ in range(NBUF)]

    flops = (2 * n_nodes * din * HPW + 2 * e_pad * de * HPW + 4 * e_pad * HPW
             + 3 * 2 * n_nodes * HPW * HPW + 2 * g * n_nodes * HPW
             + 2 * 2 * g * HPW * HPW)
    bytes_accessed = 4 * (x.size + ea.size + batch2.size + 2 * e_pad
                          + wmx.size + wme.size + bm.size
                          + 5 * HPW * HPW + 5 * HPW + g * HPW)

    out = pl.pallas_call(
        _make_body(n_steps, tps, te, n_nodes, g),
        out_shape=jax.ShapeDtypeStruct((g, HPW), f32),
        grid_spec=pltpu.PrefetchScalarGridSpec(
            num_scalar_prefetch=2,
            grid=(n_steps,),
            in_specs=in_specs,
            out_specs=pl.BlockSpec((g, HPW), lambda t, *_: (0, 0)),
            scratch_shapes=scratch,
        ),
        compiler_params=pltpu.CompilerParams(
            dimension_semantics=("arbitrary",),
            vmem_limit_bytes=64 * 1024 * 1024,
        ),
        cost_estimate=pl.CostEstimate(flops=int(flops), transcendentals=0,
                                      bytes_accessed=int(bytes_accessed)),
    )(src, dst,
      x, ea, batch2,
      wmx, wme, bm, w1, b1, w2, b2, w3, b3, w4, b4, w5, b5)

    return out[:, :1]


def kernel(node_attr, edge_attr, edge_index, batch,
           wmx, wme, bm, w1, b1, w2, b2, w3, b3, w4, b4, w5, b5):
    params = dict(wmx=wmx, wme=wme, bm=bm,
                  w1=w1, b1=b1, w2=w2, b2=b2, w3=w3, b3=b3,
                  w4=w4, b4=b4, w5=w5, b5=b5)
    return _forward(node_attr, edge_attr, edge_index, batch, params,
                    num_graphs=128)


# speedup vs baseline: 1.7816x; 1.7816x over previous
import jax
import jax.numpy as jnp
from jax.experimental import pallas as pl
from jax.experimental.pallas import tpu as pltpu

HPW = 128
NBUF = 8
GRP = 8


def _round_up(v, m):
    return ((v + m - 1) // m) * m


def _pad_2d(a, rows, cols):
    out = jnp.zeros((rows, cols), jnp.float32)
    return out.at[: a.shape[0], : a.shape[1]].set(a.astype(jnp.float32))


def _make_body(n_steps, tiles_per_step, tile_e, n_nodes, n_graphs):
    f32 = jnp.float32

    def body(
        sd_ref,
        x_ref,
        ea_ref,
        batch_ref,
        wmx_ref, wme_ref, bm_ref,
        w1_ref, b1_ref, w2_ref, b2_ref, w3_ref, b3_ref,
        w4_ref, b4_ref, w5_ref, b5_ref,
        out_ref,
        xw3_ref,
        eaw_ref,
        *hbufs,
    ):
        t = pl.program_id(0)

        @pl.when(t == 0)
        def _init():
            xw = jnp.dot(x_ref[...], wmx_ref[...], preferred_element_type=f32)
            xw3_ref[...] = xw.reshape(xw3_ref.shape)
            for hb in hbufs:
                hb[...] = jnp.zeros_like(hb)

        def tile_body(k, carry):
            ea_tile = ea_ref[pl.ds(pl.multiple_of(k * tile_e, tile_e), tile_e), :]
            eaw_ref[...] = (
                jnp.dot(ea_tile, wme_ref[...], preferred_element_type=f32)
                + bm_ref[...]
            )

            base = (t * tiles_per_step + k) * tile_e

            for g in range(tile_e // GRP):
                msgs = []
                dsts = []
                for j in range(GRP):
                    e = base + g * GRP + j
                    p = sd_ref[e]
                    s = jnp.bitwise_and(p, 16383)
                    dsts.append(jax.lax.shift_right_logical(p, 14))
                    xrow = xw3_ref[s, 0]
                    erow = eaw_ref[g * GRP + j]
                    msgs.append(jnp.maximum(xrow + erow, 0.0))
                loads = [hbufs[j][dsts[j], 0] for j in range(GRP)]
                for j in range(GRP):
                    hbufs[j][dsts[j], 0] = loads[j] + msgs[j]
            return carry

        jax.lax.fori_loop(0, tiles_per_step, tile_body, 0)

        @pl.when(t == n_steps - 1)
        def _finalize():
            acc = hbufs[0][...]
            for hb in hbufs[1:]:
                acc = acc + hb[...]
            h = acc.reshape(acc.shape[0], HPW)[:n_nodes]
            h = jnp.maximum(
                jnp.dot(h, w1_ref[...], preferred_element_type=f32) + b1_ref[...], 0.0)
            h = jnp.maximum(
                jnp.dot(h, w2_ref[...], preferred_element_type=f32) + b2_ref[...], 0.0)
            h = jnp.maximum(
                jnp.dot(h, w3_ref[...], preferred_element_type=f32) + b3_ref[...], 0.0)

            gid = jax.lax.broadcasted_iota(jnp.int32, (n_graphs, n_nodes), 0)
            pool = (gid == batch_ref[...]).astype(f32)
            gvec = jnp.dot(pool, h, preferred_element_type=f32)

            gvec = jnp.maximum(
                jnp.dot(gvec, w4_ref[...], preferred_element_type=f32) + b4_ref[...], 0.0)
            out_ref[...] = (
                jnp.dot(gvec, w5_ref[...], preferred_element_type=f32) + b5_ref[...])

    return body


def _forward(node_attr, edge_attr, edge_index, batch, params, num_graphs,
             tile_e=512, tiles_per_step=16):
    f32 = jnp.float32
    n_nodes, din = node_attr.shape
    n_edges, de = edge_attr.shape
    g = int(num_graphs)

    te = max(GRP, _round_up(min(tile_e, n_edges), GRP))
    tps = tiles_per_step
    e_pad = _round_up(n_edges, te * tps)
    n_steps = e_pad // (te * tps)

    assert n_nodes < (1 << 14), "packed src/dst indexing assumes < 16384 nodes"
    x = node_attr.astype(f32)
    ea = jnp.zeros((e_pad, de), f32).at[:n_edges].set(edge_attr.astype(f32))
    src = jnp.zeros((e_pad,), jnp.int32).at[:n_edges].set(
        edge_index[0].astype(jnp.int32))
    dst = jnp.full((e_pad,), n_nodes, jnp.int32).at[:n_edges].set(
        edge_index[1].astype(jnp.int32))
    sd = jnp.bitwise_or(src, jnp.left_shift(dst, 14))
    batch2 = batch.astype(jnp.int32).reshape(1, n_nodes)

    wmx = _pad_2d(params["wmx"], din, HPW)
    wme = _pad_2d(params["wme"], de, HPW)
    bm = _pad_2d(params["bm"], 1, HPW)
    w1 = _pad_2d(params["w1"], HPW, HPW); b1 = _pad_2d(params["b1"], 1, HPW)
    w2 = _pad_2d(params["w2"], HPW, HPW); b2 = _pad_2d(params["b2"], 1, HPW)
    w3 = _pad_2d(params["w3"], HPW, HPW); b3 = _pad_2d(params["b3"], 1, HPW)
    w4 = _pad_2d(params["w4"], HPW, HPW); b4 = _pad_2d(params["b4"], 1, HPW)
    w5 = _pad_2d(params["w5"], HPW, HPW); b5 = _pad_2d(params["b5"], 1, HPW)

    def cspec(shape):
        return pl.BlockSpec(shape, lambda t, *_: tuple(0 for _ in shape))

    in_specs = [
        cspec((n_nodes, din)),
        pl.BlockSpec((te * tps, de), lambda t, *_: (t, 0)),
        cspec((1, n_nodes)),
        cspec((din, HPW)), cspec((de, HPW)), cspec((1, HPW)),
        cspec((HPW, HPW)), cspec((1, HPW)),
        cspec((HPW, HPW)), cspec((1, HPW)),
        cspec((HPW, HPW)), cspec((1, HPW)),
        cspec((HPW, HPW)), cspec((1, HPW)),
        cspec((HPW, HPW)), cspec((1, HPW)),
    ]

    scratch = [
        pltpu.VMEM((n_nodes, 1, HPW), f32),
        pltpu.VMEM((te, HPW), f32),
    ] + [pltpu.VMEM((n_nodes + 8, 1, HPW), f32) for _ in range(NBUF)]

    flops = (2 * n_nodes * din * HPW + 2 * e_pad * de * HPW + 4 * e_pad * HPW
             + 3 * 2 * n_nodes * HPW * HPW + 2 * g * n_nodes * HPW
             + 2 * 2 * g * HPW * HPW)
    bytes_accessed = 4 * (x.size + ea.size + batch2.size + 2 * e_pad
                          + wmx.size + wme.size + bm.size
                          + 5 * HPW * HPW + 5 * HPW + g * HPW)

    out = pl.pallas_call(
        _make_body(n_steps, tps, te, n_nodes, g),
        out_shape=jax.ShapeDtypeStruct((g, HPW), f32),
        grid_spec=pltpu.PrefetchScalarGridSpec(
            num_scalar_prefetch=1,
            grid=(n_steps,),
            in_specs=in_specs,
            out_specs=pl.BlockSpec((g, HPW), lambda t, *_: (0, 0)),
            scratch_shapes=scratch,
        ),
        compiler_params=pltpu.CompilerParams(
            dimension_semantics=("arbitrary",),
            vmem_limit_bytes=64 * 1024 * 1024,
        ),
        cost_estimate=pl.CostEstimate(flops=int(flops), transcendentals=0,
                                      bytes_accessed=int(bytes_accessed)),
    )(sd,
      x, ea, batch2,
      wmx, wme, bm, w1, b1, w2, b2, w3, b3, w4, b4, w5, b5)

    return out[:, :1]


def kernel(node_attr, edge_attr, edge_index, batch,
           wmx, wme, bm, w1, b1, w2, b2, w3, b3, w4, b4, w5, b5):
    params = dict(wmx=wmx, wme=wme, bm=bm,
                  w1=w1, b1=b1, w2=w2, b2=b2, w3=w3, b3=b3,
                  w4=w4, b4=b4, w5=w5, b5=b5)
    return _forward(node_attr, edge_attr, edge_index, batch, params,
                    num_graphs=128)


# speedup vs baseline: 2.1633x; 1.2143x over previous
import jax
import jax.numpy as jnp
from jax.experimental import pallas as pl
from jax.experimental.pallas import tpu as pltpu

HPW = 128
NBUF = 4
GRP = 4


def _round_up(v, m):
    return ((v + m - 1) // m) * m


def _pad_2d(a, rows, cols):
    out = jnp.zeros((rows, cols), jnp.float32)
    return out.at[: a.shape[0], : a.shape[1]].set(a.astype(jnp.float32))


def _make_body(n_steps, tiles_per_step, tile_e, n_nodes, n_graphs):
    f32 = jnp.float32

    def body(
        sd_ref,
        x_ref,
        ea_ref,
        batch_ref,
        wmx_ref, wme_ref, bm_ref,
        w1_ref, b1_ref, w2_ref, b2_ref, w3_ref, b3_ref,
        w4_ref, b4_ref, w5_ref, b5_ref,
        out_ref,
        xw3_ref,
        eaw_ref,
        *hbufs,
    ):
        t = pl.program_id(0)

        @pl.when(t == 0)
        def _init():
            xw = jnp.dot(x_ref[...], wmx_ref[...], preferred_element_type=f32)
            xw3_ref[...] = xw.reshape(xw3_ref.shape)
            for hb in hbufs:
                hb[...] = jnp.zeros_like(hb)

        def tile_body(k, carry):
            ea_tile = ea_ref[pl.ds(pl.multiple_of(k * tile_e, tile_e), tile_e), :]
            eaw_ref[...] = (
                jnp.dot(ea_tile, wme_ref[...], preferred_element_type=f32)
                + bm_ref[...]
            )

            base = (t * tiles_per_step + k) * tile_e

            for g in range(tile_e // GRP):
                msgs = []
                dsts = []
                for j in range(GRP):
                    e = base + g * GRP + j
                    p = sd_ref[e]
                    s = jnp.bitwise_and(p, 16383)
                    dsts.append(jax.lax.shift_right_logical(p, 14))
                    xrow = xw3_ref[s, 0]
                    erow = eaw_ref[g * GRP + j]
                    msgs.append(jnp.maximum(xrow + erow, 0.0))
                loads = [hbufs[j][dsts[j], 0] for j in range(GRP)]
                for j in range(GRP):
                    hbufs[j][dsts[j], 0] = loads[j] + msgs[j]
            return carry

        jax.lax.fori_loop(0, tiles_per_step, tile_body, 0)

        @pl.when(t == n_steps - 1)
        def _finalize():
            acc = hbufs[0][...]
            for hb in hbufs[1:]:
                acc = acc + hb[...]
            h = acc.reshape(acc.shape[0], HPW)[:n_nodes]
            h = jnp.maximum(
                jnp.dot(h, w1_ref[...], preferred_element_type=f32) + b1_ref[...], 0.0)
            h = jnp.maximum(
                jnp.dot(h, w2_ref[...], preferred_element_type=f32) + b2_ref[...], 0.0)
            h = jnp.maximum(
                jnp.dot(h, w3_ref[...], preferred_element_type=f32) + b3_ref[...], 0.0)

            gid = jax.lax.broadcasted_iota(jnp.int32, (n_graphs, n_nodes), 0)
            pool = (gid == batch_ref[...]).astype(f32)
            gvec = jnp.dot(pool, h, preferred_element_type=f32)

            gvec = jnp.maximum(
                jnp.dot(gvec, w4_ref[...], preferred_element_type=f32) + b4_ref[...], 0.0)
            out_ref[...] = (
                jnp.dot(gvec, w5_ref[...], preferred_element_type=f32) + b5_ref[...])

    return body


def _forward(node_attr, edge_attr, edge_index, batch, params, num_graphs,
             tile_e=512, tiles_per_step=16):
    f32 = jnp.float32
    n_nodes, din = node_attr.shape
    n_edges, de = edge_attr.shape
    g = int(num_graphs)

    te = max(GRP, _round_up(min(tile_e, n_edges), GRP))
    tps = tiles_per_step
    e_pad = _round_up(n_edges, te * tps)
    n_steps = e_pad // (te * tps)

    assert n_nodes < (1 << 14), "packed src/dst indexing assumes < 16384 nodes"
    x = node_attr.astype(f32)
    ea = jnp.zeros((e_pad, de), f32).at[:n_edges].set(edge_attr.astype(f32))
    src = jnp.zeros((e_pad,), jnp.int32).at[:n_edges].set(
        edge_index[0].astype(jnp.int32))
    dst = jnp.full((e_pad,), n_nodes, jnp.int32).at[:n_edges].set(
        edge_index[1].astype(jnp.int32))
    sd = jnp.bitwise_or(src, jnp.left_shift(dst, 14))
    batch2 = batch.astype(jnp.int32).reshape(1, n_nodes)

    wmx = _pad_2d(params["wmx"], din, HPW)
    wme = _pad_2d(params["wme"], de, HPW)
    bm = _pad_2d(params["bm"], 1, HPW)
    w1 = _pad_2d(params["w1"], HPW, HPW); b1 = _pad_2d(params["b1"], 1, HPW)
    w2 = _pad_2d(params["w2"], HPW, HPW); b2 = _pad_2d(params["b2"], 1, HPW)
    w3 = _pad_2d(params["w3"], HPW, HPW); b3 = _pad_2d(params["b3"], 1, HPW)
    w4 = _pad_2d(params["w4"], HPW, HPW); b4 = _pad_2d(params["b4"], 1, HPW)
    w5 = _pad_2d(params["w5"], HPW, HPW); b5 = _pad_2d(params["b5"], 1, HPW)

    def cspec(shape):
        return pl.BlockSpec(shape, lambda t, *_: tuple(0 for _ in shape))

    in_specs = [
        cspec((n_nodes, din)),
        pl.BlockSpec((te * tps, de), lambda t, *_: (t, 0)),
        cspec((1, n_nodes)),
        cspec((din, HPW)), cspec((de, HPW)), cspec((1, HPW)),
        cspec((HPW, HPW)), cspec((1, HPW)),
        cspec((HPW, HPW)), cspec((1, HPW)),
        cspec((HPW, HPW)), cspec((1, HPW)),
        cspec((HPW, HPW)), cspec((1, HPW)),
        cspec((HPW, HPW)), cspec((1, HPW)),
    ]

    scratch = [
        pltpu.VMEM((n_nodes, 1, HPW), f32),
        pltpu.VMEM((te, HPW), f32),
    ] + [pltpu.VMEM((n_nodes + 8, 1, HPW), f32) for _ in range(NBUF)]

    flops = (2 * n_nodes * din * HPW + 2 * e_pad * de * HPW + 4 * e_pad * HPW
             + 3 * 2 * n_nodes * HPW * HPW + 2 * g * n_nodes * HPW
             + 2 * 2 * g * HPW * HPW)
    bytes_accessed = 4 * (x.size + ea.size + batch2.size + 2 * e_pad
                          + wmx.size + wme.size + bm.size
                          + 5 * HPW * HPW + 5 * HPW + g * HPW)

    out = pl.pallas_call(
        _make_body(n_steps, tps, te, n_nodes, g),
        out_shape=jax.ShapeDtypeStruct((g, HPW), f32),
        grid_spec=pltpu.PrefetchScalarGridSpec(
            num_scalar_prefetch=1,
            grid=(n_steps,),
            in_specs=in_specs,
            out_specs=pl.BlockSpec((g, HPW), lambda t, *_: (0, 0)),
            scratch_shapes=scratch,
        ),
        compiler_params=pltpu.CompilerParams(
            dimension_semantics=("arbitrary",),
            vmem_limit_bytes=64 * 1024 * 1024,
        ),
        cost_estimate=pl.CostEstimate(flops=int(flops), transcendentals=0,
                                      bytes_accessed=int(bytes_accessed)),
    )(sd,
      x, ea, batch2,
      wmx, wme, bm, w1, b1, w2, b2, w3, b3, w4, b4, w5, b5)

    return out[:, :1]


def kernel(node_attr, edge_attr, edge_index, batch,
           wmx, wme, bm, w1, b1, w2, b2, w3, b3, w4, b4, w5, b5):
    params = dict(wmx=wmx, wme=wme, bm=bm,
                  w1=w1, b1=b1, w2=w2, b2=b2, w3=w3, b3=b3,
                  w4=w4, b4=b4, w5=w5, b5=b5)
    return _forward(node_attr, edge_attr, edge_index, batch, params,
                    num_graphs=128)
